# Initial kernel scaffold; baseline (speedup 1.0000x reference)
import jax
import jax.numpy as jnp
from jax.experimental import pallas as pl


def kernel(x, edge_index, edge_attr, batch, ecfp, word300_mean, torsion, node_mask, edge_mask, W1, b1, W2, b2, gn1_alpha, gn1_gamma, gn1_beta, gn2_alpha, gn2_gamma, gn2_beta, Wdec, bdec, Wfc1, bfc1, Wfc2, bfc2, Wecfp, becfp, Wtor, btor, Ww300, bw300):
    raise NotImplementedError("write your pallas kernel here")


# speedup vs baseline: 1.0011x; 1.0011x over previous
import jax
import jax.numpy as jnp
from jax.experimental import pallas as pl
from jax.experimental.pallas import tpu as pltpu

N = 10000
E = 320000
D = 128
H = 128
G = 512
L = 4


def _mask_kernel(x_ref, m_ref, o_ref):
    o_ref[...] = jnp.where(m_ref[...], 0.0, x_ref[...])


def _graph_norm(z, batch, alpha, gamma, beta, cnt):
    mean = jax.ops.segment_sum(z, batch, num_segments=G) / cnt[:, None]
    out = z - alpha * mean[batch]
    var = jax.ops.segment_sum(out * out, batch, num_segments=G) / cnt[:, None]
    std = jnp.sqrt(var + 1e-5)
    return out / std[batch] * gamma + beta


def kernel(x, edge_index, edge_attr, batch, ecfp, word300_mean, torsion, node_mask, edge_mask, W1, b1, W2, b2, gn1_alpha, gn1_gamma, gn1_beta, gn2_alpha, gn2_gamma, gn2_beta, Wdec, bdec, Wfc1, bfc1, Wfc2, bfc2, Wecfp, becfp, Wtor, btor, Ww300, bw300):
    x_masked = pl.pallas_call(
        _mask_kernel,
        out_shape=jax.ShapeDtypeStruct((N, D), jnp.float32),
    )(x, node_mask)
    src = edge_index[0]
    dst = edge_index[1]
    counts = jax.ops.segment_sum(jnp.ones((N,), jnp.float32), batch, num_segments=G)
    cnt = jnp.maximum(counts, 1.0)
    h = x_masked
    for l in range(L):
        agg = jax.ops.segment_sum(h[src], dst, num_segments=N)
        z = h + agg
        z = z @ W1[l] + b1[l]
        z = _graph_norm(z, batch, gn1_alpha[l], gn1_gamma[l], gn1_beta[l], cnt)
        z = jax.nn.leaky_relu(z, 0.01)
        z = z @ W2[l] + b2[l]
        if l < L - 1:
            z = _graph_norm(z, batch, gn2_alpha[l], gn2_gamma[l], gn2_beta[l], cnt)
            z = jax.nn.leaky_relu(z, 0.01)
        h = z
    recon = h @ Wdec + bdec
    m = node_mask.astype(jnp.float32)
    pre_loss = jnp.sum(((recon - x) * m) ** 2) / jnp.maximum(jnp.sum(m), 1.0)
    mean_p = jax.ops.segment_sum(h, batch, num_segments=G) / cnt[:, None]
    add_p = jax.ops.segment_sum(h, batch, num_segments=G)
    max_p = jax.ops.segment_max(h, batch, num_segments=G)
    max_p = jnp.where(counts[:, None] > 0, max_p, 0.0)
    pooled = jnp.concatenate([mean_p, add_p, max_p], axis=1)
    pooled = jax.nn.leaky_relu(pooled @ Wfc1 + bfc1, 0.01)
    class_out = (pooled @ Wfc2 + bfc2) / 2.0
    ecfp_loss = jnp.mean((pooled @ Wecfp + becfp - ecfp) ** 2)
    tor_loss = jnp.mean((pooled @ Wtor + btor - torsion) ** 2)
    w300_loss = jnp.mean((pooled @ Ww300 + bw300 - word300_mean) ** 2)
    return class_out, pre_loss + ecfp_loss + tor_loss + w300_loss


# speedup vs baseline: 1.7203x; 1.7184x over previous
import functools

import jax
import jax.numpy as jnp
from jax import lax
from jax.experimental import pallas as pl
from jax.experimental.pallas import tpu as pltpu
from jax.experimental.pallas import tpu_sc as plsc

N = 10000
E = 320000
D = 128
H = 128
G = 512
L = 4

NC = 2
NS = 16
NW = NC * NS
NP = 10240
RPT = NP // NS
EPT = 10240
EPAD = NW * EPT
CH = 128
NCHUNK = EPT // CH


def _edge_agg_body(h_hbm, srcp_hbm, dstp_hbm, zrows_hbm, out_hbm,
                   sidx, didx, rows, agg_sh, sem):
    c = lax.axis_index("c")
    s = lax.axis_index("s")
    wid = s * NC + c
    pltpu.sync_copy(zrows_hbm, agg_sh.at[pl.ds(s * RPT, RPT)])
    plsc.subcore_barrier()
    ebase = wid * EPT

    def chunk(i, carry):
        off = ebase + i * CH
        pltpu.sync_copy(srcp_hbm.at[pl.ds(off, CH)], sidx)
        pltpu.sync_copy(dstp_hbm.at[pl.ds(off, CH)], didx.at[0])
        pltpu.async_copy(h_hbm.at[sidx], rows, sem).wait()
        pltpu.sync_copy(rows, agg_sh.at[didx.at[0]], add=True)
        return carry

    lax.fori_loop(0, NCHUNK, chunk, 0)
    plsc.subcore_barrier()
    pltpu.sync_copy(agg_sh.at[pl.ds(s * RPT, RPT)],
                    out_hbm.at[c, pl.ds(s * RPT, RPT)])


_edge_agg_call = functools.partial(
    pl.kernel,
    out_type=jax.ShapeDtypeStruct((NC, NP, D), jnp.float32),
    mesh=plsc.VectorSubcoreMesh(core_axis_name="c", subcore_axis_name="s"),
    scratch_types=[
        pltpu.VMEM((CH,), jnp.int32),
        pltpu.VMEM((1, CH), jnp.int32),
        pltpu.VMEM((CH, D), jnp.float32),
        pltpu.VMEM_SHARED((NP, D), jnp.float32),
        pltpu.SemaphoreType.DMA,
    ],
)(_edge_agg_body)


def _edge_agg(h, src_p, dst_p, zrows):
    parts = _edge_agg_call(h, src_p, dst_p, zrows)
    return parts[0, :N] + parts[1, :N]


def _mask_kernel(x_ref, m_ref, o_ref):
    o_ref[...] = jnp.where(m_ref[...], 0.0, x_ref[...])


def _graph_norm(z, batch, alpha, gamma, beta, cnt):
    mean = jax.ops.segment_sum(z, batch, num_segments=G) / cnt[:, None]
    out = z - alpha * mean[batch]
    var = jax.ops.segment_sum(out * out, batch, num_segments=G) / cnt[:, None]
    std = jnp.sqrt(var + 1e-5)
    return out / std[batch] * gamma + beta


def kernel(x, edge_index, edge_attr, batch, ecfp, word300_mean, torsion, node_mask, edge_mask, W1, b1, W2, b2, gn1_alpha, gn1_gamma, gn1_beta, gn2_alpha, gn2_gamma, gn2_beta, Wdec, bdec, Wfc1, bfc1, Wfc2, bfc2, Wecfp, becfp, Wtor, btor, Ww300, bw300):
    x_masked = pl.pallas_call(
        _mask_kernel,
        out_shape=jax.ShapeDtypeStruct((N, D), jnp.float32),
    )(x, node_mask)
    src = edge_index[0]
    dst = edge_index[1]
    pad = EPAD - E
    src_p = jnp.concatenate([src, jnp.zeros((pad,), jnp.int32)])
    dst_p = jnp.concatenate([dst, jnp.full((pad,), N, jnp.int32)])
    zrows = jnp.zeros((RPT, D), jnp.float32)
    counts = jax.ops.segment_sum(jnp.ones((N,), jnp.float32), batch, num_segments=G)
    cnt = jnp.maximum(counts, 1.0)
    h = x_masked
    for l in range(L):
        agg = _edge_agg(h, src_p, dst_p, zrows)
        z = h + agg
        z = z @ W1[l] + b1[l]
        z = _graph_norm(z, batch, gn1_alpha[l], gn1_gamma[l], gn1_beta[l], cnt)
        z = jax.nn.leaky_relu(z, 0.01)
        z = z @ W2[l] + b2[l]
        if l < L - 1:
            z = _graph_norm(z, batch, gn2_alpha[l], gn2_gamma[l], gn2_beta[l], cnt)
            z = jax.nn.leaky_relu(z, 0.01)
        h = z
    recon = h @ Wdec + bdec
    m = node_mask.astype(jnp.float32)
    pre_loss = jnp.sum(((recon - x) * m) ** 2) / jnp.maximum(jnp.sum(m), 1.0)
    mean_p = jax.ops.segment_sum(h, batch, num_segments=G) / cnt[:, None]
    add_p = jax.ops.segment_sum(h, batch, num_segments=G)
    max_p = jax.ops.segment_max(h, batch, num_segments=G)
    max_p = jnp.where(counts[:, None] > 0, max_p, 0.0)
    pooled = jnp.concatenate([mean_p, add_p, max_p], axis=1)
    pooled = jax.nn.leaky_relu(pooled @ Wfc1 + bfc1, 0.01)
    class_out = (pooled @ Wfc2 + bfc2) / 2.0
    ecfp_loss = jnp.mean((pooled @ Wecfp + becfp - ecfp) ** 2)
    tor_loss = jnp.mean((pooled @ Wtor + btor - torsion) ** 2)
    w300_loss = jnp.mean((pooled @ Ww300 + bw300 - word300_mean) ** 2)
    return class_out, pre_loss + ecfp_loss + tor_loss + w300_loss


# speedup vs baseline: 1.8532x; 1.0773x over previous
import functools

import jax
import jax.numpy as jnp
from jax import lax
from jax.experimental import pallas as pl
from jax.experimental.pallas import tpu as pltpu
from jax.experimental.pallas import tpu_sc as plsc

N = 10000
E = 320000
D = 128
H = 128
G = 512
L = 4

NC = 2
NS = 16
NW = NC * NS
NP = 10112
RPT = NP // NS
EPT = 10240
EPAD = NW * EPT
CH = 128
NCHUNK = EPT // CH


K = 2
IB = 16
NB = NCHUNK // IB
NG = IB // K


def _edge_agg_body(h_hbm, srcp_hbm, dstp_hbm, zrows_hbm, out_hbm,
                   sidx, didx, rows, agg_sh, sem):
    c = lax.axis_index("c")
    s = lax.axis_index("s")
    wid = s * NC + c
    pltpu.sync_copy(zrows_hbm, agg_sh.at[pl.ds(s * RPT, RPT)])
    plsc.subcore_barrier()

    def block(b, carry):
        off = wid * EPT + b * (IB * CH)
        pltpu.sync_copy(srcp_hbm.at[pl.ds(off, IB * CH)], sidx)
        pltpu.sync_copy(dstp_hbm.at[wid, pl.ds(b * IB, IB)], didx)

        def group(g, carry2):
            base = g * K
            cps = [
                pltpu.async_copy(
                    h_hbm.at[sidx.at[pl.ds((base + k) * CH, CH)]],
                    rows.at[k], sem)
                for k in range(K)
            ]
            for cp in cps:
                cp.wait()
            for k in range(K):
                pltpu.sync_copy(rows.at[k], agg_sh.at[didx.at[base + k]],
                                add=True)
            return carry2

        return lax.fori_loop(0, NG, group, carry)

    lax.fori_loop(0, NB, block, 0)
    plsc.subcore_barrier()
    pltpu.sync_copy(agg_sh.at[pl.ds(s * RPT, RPT)],
                    out_hbm.at[c, pl.ds(s * RPT, RPT)])


_edge_agg_call = functools.partial(
    pl.kernel,
    out_type=jax.ShapeDtypeStruct((NC, NP, D), jnp.float32),
    mesh=plsc.VectorSubcoreMesh(core_axis_name="c", subcore_axis_name="s"),
    scratch_types=[
        pltpu.VMEM((IB * CH,), jnp.int32),
        pltpu.VMEM((IB, CH), jnp.int32),
        pltpu.VMEM((K, CH, D), jnp.float32),
        pltpu.VMEM_SHARED((NP, D), jnp.float32),
        pltpu.SemaphoreType.DMA,
    ],
)(_edge_agg_body)


def _edge_agg(h, src_p, dst_p, zrows):
    parts = _edge_agg_call(h, src_p, dst_p, zrows)
    return parts[0, :N] + parts[1, :N]


def _mask_kernel(x_ref, m_ref, o_ref):
    o_ref[...] = jnp.where(m_ref[...], 0.0, x_ref[...])


def _graph_norm(z, batch, alpha, gamma, beta, cnt):
    mean = jax.ops.segment_sum(z, batch, num_segments=G) / cnt[:, None]
    out = z - alpha * mean[batch]
    var = jax.ops.segment_sum(out * out, batch, num_segments=G) / cnt[:, None]
    std = jnp.sqrt(var + 1e-5)
    return out / std[batch] * gamma + beta


def kernel(x, edge_index, edge_attr, batch, ecfp, word300_mean, torsion, node_mask, edge_mask, W1, b1, W2, b2, gn1_alpha, gn1_gamma, gn1_beta, gn2_alpha, gn2_gamma, gn2_beta, Wdec, bdec, Wfc1, bfc1, Wfc2, bfc2, Wecfp, becfp, Wtor, btor, Ww300, bw300):
    x_masked = pl.pallas_call(
        _mask_kernel,
        out_shape=jax.ShapeDtypeStruct((N, D), jnp.float32),
    )(x, node_mask)
    src = edge_index[0]
    dst = edge_index[1]
    pad = EPAD - E
    src_p = jnp.concatenate([src, jnp.zeros((pad,), jnp.int32)])
    dst_p = jnp.concatenate([dst, jnp.full((pad,), N, jnp.int32)]).reshape(NW, NCHUNK, CH)
    zrows = jnp.zeros((RPT, D), jnp.float32)
    counts = jax.ops.segment_sum(jnp.ones((N,), jnp.float32), batch, num_segments=G)
    cnt = jnp.maximum(counts, 1.0)
    h = x_masked
    for l in range(L):
        agg = _edge_agg(h, src_p, dst_p, zrows)
        z = h + agg
        z = z @ W1[l] + b1[l]
        z = _graph_norm(z, batch, gn1_alpha[l], gn1_gamma[l], gn1_beta[l], cnt)
        z = jax.nn.leaky_relu(z, 0.01)
        z = z @ W2[l] + b2[l]
        if l < L - 1:
            z = _graph_norm(z, batch, gn2_alpha[l], gn2_gamma[l], gn2_beta[l], cnt)
            z = jax.nn.leaky_relu(z, 0.01)
        h = z
    recon = h @ Wdec + bdec
    m = node_mask.astype(jnp.float32)
    pre_loss = jnp.sum(((recon - x) * m) ** 2) / jnp.maximum(jnp.sum(m), 1.0)
    mean_p = jax.ops.segment_sum(h, batch, num_segments=G) / cnt[:, None]
    add_p = jax.ops.segment_sum(h, batch, num_segments=G)
    max_p = jax.ops.segment_max(h, batch, num_segments=G)
    max_p = jnp.where(counts[:, None] > 0, max_p, 0.0)
    pooled = jnp.concatenate([mean_p, add_p, max_p], axis=1)
    pooled = jax.nn.leaky_relu(pooled @ Wfc1 + bfc1, 0.01)
    class_out = (pooled @ Wfc2 + bfc2) / 2.0
    ecfp_loss = jnp.mean((pooled @ Wecfp + becfp - ecfp) ** 2)
    tor_loss = jnp.mean((pooled @ Wtor + btor - torsion) ** 2)
    w300_loss = jnp.mean((pooled @ Ww300 + bw300 - word300_mean) ** 2)
    return class_out, pre_loss + ecfp_loss + tor_loss + w300_loss


# speedup vs baseline: 2.3530x; 1.2697x over previous
import functools

import jax
import jax.numpy as jnp
from jax import lax
from jax.experimental import pallas as pl
from jax.experimental.pallas import tpu as pltpu
from jax.experimental.pallas import tpu_sc as plsc

N = 10000
E = 320000
D = 128
H = 128
G = 512
L = 4

NC = 2
NS = 16
NW = NC * NS
NP = 10112
RPT = NP // NS
EPT = 10240
EPAD = NW * EPT
CH = 128
NCHUNK = EPT // CH


K = 2
IB = 16
NB = NCHUNK // IB
NG = IB // K


def _edge_agg_body(h_hbm, srcp_hbm, dstp_hbm, zrows_hbm, out_hbm,
                   sidx, didx, rows, agg_sh, sem):
    c = lax.axis_index("c")
    s = lax.axis_index("s")
    wid = s * NC + c
    pltpu.sync_copy(zrows_hbm, agg_sh.at[pl.ds(s * RPT, RPT)])
    plsc.subcore_barrier()

    def block(b, carry):
        off = wid * EPT + b * (IB * CH)
        pltpu.sync_copy(srcp_hbm.at[pl.ds(off, IB * CH)], sidx)
        pltpu.sync_copy(dstp_hbm.at[wid, pl.ds(b * IB, IB)], didx)

        def group(g, carry2):
            base = g * K
            cps = [
                pltpu.async_copy(
                    h_hbm.at[sidx.at[pl.ds((base + k) * CH, CH)]],
                    rows.at[k], sem)
                for k in range(K)
            ]
            for cp in cps:
                cp.wait()
            for k in range(K):
                pltpu.sync_copy(rows.at[k], agg_sh.at[didx.at[base + k]],
                                add=True)
            return carry2

        return lax.fori_loop(0, NG, group, carry)

    lax.fori_loop(0, NB, block, 0)
    plsc.subcore_barrier()
    pltpu.sync_copy(agg_sh.at[pl.ds(s * RPT, RPT)],
                    out_hbm.at[c, pl.ds(s * RPT, RPT)])


_edge_agg_call = functools.partial(
    pl.kernel,
    out_type=jax.ShapeDtypeStruct((NC, NP, D), jnp.float32),
    mesh=plsc.VectorSubcoreMesh(core_axis_name="c", subcore_axis_name="s"),
    scratch_types=[
        pltpu.VMEM((IB * CH,), jnp.int32),
        pltpu.VMEM((IB, CH), jnp.int32),
        pltpu.VMEM((K, CH, D), jnp.float32),
        pltpu.VMEM_SHARED((NP, D), jnp.float32),
        pltpu.SemaphoreType.DMA,
    ],
)(_edge_agg_body)


NBLK = NP // CH
PAD_GID = 520


def _mask_kernel(x_ref, m_ref, h0_ref, xp_ref, mf_ref):
    mf = m_ref[...].astype(jnp.float32)
    h0_ref[pl.ds(0, N), :] = jnp.where(m_ref[...], 0.0, x_ref[...])
    h0_ref[pl.ds(N, NP - N), :] = jnp.zeros((NP - N, D), jnp.float32)
    xp_ref[pl.ds(0, N), :] = x_ref[...]
    xp_ref[pl.ds(N, NP - N), :] = jnp.zeros((NP - N, D), jnp.float32)
    mf_ref[pl.ds(0, N), :] = mf
    mf_ref[pl.ds(N, NP - N), :] = jnp.zeros((NP - N, D), jnp.float32)


_GIDS = None


def _dot(a, b):
    return jax.lax.dot_general(a, b, (((1,), (0,)), ((), ())),
                               precision=jax.lax.Precision.HIGHEST,
                               preferred_element_type=jnp.float32)


def _gn_inplace(z_ref, brow_ref, bcol_ref, s12_ref, st_ref, cnt_ref,
                alpha, gamma, beta):
    s12_ref[...] = jnp.zeros_like(s12_ref)
    cnt_ref[...] = jnp.zeros_like(cnt_ref)
    gl_col = jax.lax.broadcasted_iota(jnp.int32, (G, 1), 0)
    gl_row = jax.lax.broadcasted_iota(jnp.int32, (1, G), 1)

    def red(b, carry):
        rb = b * CH
        zb = z_ref[pl.ds(rb, CH), :]
        bb = brow_ref[pl.ds(b, 1), :]
        A = (gl_col == bb).astype(jnp.float32)
        zcat = jnp.concatenate([zb, zb * zb], axis=1)
        s12_ref[...] = s12_ref[...] + _dot(A, zcat)
        cnt_ref[...] = cnt_ref[...] + jnp.sum(A, axis=1, keepdims=True)
        return carry

    jax.lax.fori_loop(0, NBLK, red, 0)

    c = jnp.maximum(cnt_ref[...], 1.0)
    S = s12_ref[...]
    mean = S[:, :H] / c
    ez2 = S[:, H:] / c
    var = ez2 + (alpha * alpha - 2.0 * alpha) * mean * mean
    sg = gamma / jnp.sqrt(var + 1e-5)
    tg = beta - alpha * mean * sg
    st_ref[...] = jnp.concatenate([sg, tg], axis=1)

    def exp_(b, carry):
        rb = b * CH
        bc = bcol_ref[pl.ds(rb, CH), :]
        B = (bc == gl_row).astype(jnp.float32)
        E = _dot(B, st_ref[...])
        zb = z_ref[pl.ds(rb, CH), :]
        zn = zb * E[:, :H] + E[:, H:]
        z_ref[pl.ds(rb, CH), :] = jnp.maximum(zn, 0.01 * zn)
        return carry

    jax.lax.fori_loop(0, NBLK, exp_, 0)


def _layer_body_mk(last):
    def body(h_ref, parts_ref, brow_ref, bcol_ref,
             W1_ref, b1_ref, W2_ref, b2_ref,
             a1_ref, g1_ref, be1_ref, a2_ref, g2_ref, be2_ref,
             out_ref, z_ref, s12_ref, st_ref, cnt_ref):
        z_ref[...] = _dot(h_ref[...] + parts_ref[0] + parts_ref[1],
                          W1_ref[...]) + b1_ref[...]
        _gn_inplace(z_ref, brow_ref, bcol_ref, s12_ref, st_ref, cnt_ref,
                    a1_ref[...], g1_ref[...], be1_ref[...])
        out_ref[...] = _dot(z_ref[...], W2_ref[...]) + b2_ref[...]
        if not last:
            _gn_inplace(out_ref, brow_ref, bcol_ref, s12_ref, st_ref, cnt_ref,
                        a2_ref[...], g2_ref[...], be2_ref[...])
    return body


_layer_call = [
    pl.pallas_call(
        _layer_body_mk(l == L - 1),
        out_shape=jax.ShapeDtypeStruct((NP, H), jnp.float32),
        scratch_shapes=[
            pltpu.VMEM((NP, H), jnp.float32),
            pltpu.VMEM((G, 2 * H), jnp.float32),
            pltpu.VMEM((G, 2 * H), jnp.float32),
            pltpu.VMEM((G, 1), jnp.float32),
        ],
    )
    for l in range(L)
]


def _final_body(h_ref, xp_ref, mf_ref, brow_ref, bcol_ref,
                Wdec_ref, bdec_ref, Wfc1_ref, bfc1_ref, Wfc2_ref, bfc2_ref,
                Wecfp_ref, becfp_ref, Wtor_ref, btor_ref, Ww300_ref, bw300_ref,
                ecfp_ref, tor_ref, w300_ref,
                cls_ref, loss_ref, sp_ref, mx_ref, cnt_ref):
    sp_ref[...] = jnp.zeros_like(sp_ref)
    cnt_ref[...] = jnp.zeros_like(cnt_ref)
    mx_ref[...] = jnp.full_like(mx_ref, -3.0e38)
    gl_col = jax.lax.broadcasted_iota(jnp.int32, (G, 1), 0)

    def red(b, carry):
        rb = b * CH
        hb = h_ref[pl.ds(rb, CH), :]
        bb = brow_ref[pl.ds(b, 1), :]
        bc = bcol_ref[pl.ds(rb, CH), :]
        A = (gl_col == bb).astype(jnp.float32)
        sp_ref[...] = sp_ref[...] + _dot(A, hb)
        cnt_ref[...] = cnt_ref[...] + jnp.sum(A, axis=1, keepdims=True)
        m = hb
        for k in (1, 2, 4, 8, 16, 32, 64):
            shifted = jnp.concatenate([m[:k], m[:-k]], axis=0)
            prev = jnp.concatenate(
                [jnp.full((k, 1), -7, jnp.int32), bc[:-k]], axis=0)
            same = bc == prev
            m = jnp.where(same, jnp.maximum(m, shifted), m)
        nxt = jnp.concatenate([bb[:, 1:], jnp.full((1, 1), -7, jnp.int32)],
                              axis=1)
        lastm = (bb != nxt).astype(jnp.float32)
        Alast = A * lastm
        P = _dot(Alast, m)
        pres = jnp.sum(Alast, axis=1, keepdims=True) > 0.0
        mx_ref[...] = jnp.where(pres, jnp.maximum(mx_ref[...], P), mx_ref[...])
        return carry

    jax.lax.fori_loop(0, NBLK, red, 0)

    counts = cnt_ref[...]
    c = jnp.maximum(counts, 1.0)
    add_p = sp_ref[...]
    mean_p = add_p / c
    max_p = jnp.where(counts > 0.0, mx_ref[...], 0.0)

    pooled = jnp.concatenate([mean_p, add_p, max_p], axis=1)
    pooled = _dot(pooled, Wfc1_ref[...]) + bfc1_ref[...]
    pooled = jnp.maximum(pooled, 0.01 * pooled)
    cls_ref[...] = (_dot(pooled, Wfc2_ref[...]) + bfc2_ref[...]) / 2.0

    recon = _dot(h_ref[...], Wdec_ref[...]) + bdec_ref[...]
    dlt = (recon - xp_ref[...]) * mf_ref[...]
    pre_loss = jnp.sum(dlt * dlt) / jnp.maximum(jnp.sum(mf_ref[...]), 1.0)

    e1 = _dot(pooled, Wecfp_ref[...]) + becfp_ref[...] - ecfp_ref[...]
    ecfp_loss = jnp.sum(e1 * e1) / (G * 2048)
    e2 = _dot(pooled, Wtor_ref[...]) + btor_ref[...] - tor_ref[...]
    tor_loss = jnp.sum(e2 * e2) / (G * 2048)
    e3 = _dot(pooled, Ww300_ref[...]) + bw300_ref[...] - w300_ref[...]
    w300_loss = jnp.sum(e3 * e3) / (G * 300)
    loss_ref[...] = jnp.full((1, 1), 0.0) + pre_loss + ecfp_loss \
        + tor_loss + w300_loss


_final_call = pl.pallas_call(
    _final_body,
    out_shape=[
        jax.ShapeDtypeStruct((G, 3), jnp.float32),
        jax.ShapeDtypeStruct((1, 1), jnp.float32),
    ],
    scratch_shapes=[
        pltpu.VMEM((G, H), jnp.float32),
        pltpu.VMEM((G, H), jnp.float32),
        pltpu.VMEM((G, 1), jnp.float32),
    ],
)


def kernel(x, edge_index, edge_attr, batch, ecfp, word300_mean, torsion, node_mask, edge_mask, W1, b1, W2, b2, gn1_alpha, gn1_gamma, gn1_beta, gn2_alpha, gn2_gamma, gn2_beta, Wdec, bdec, Wfc1, bfc1, Wfc2, bfc2, Wecfp, becfp, Wtor, btor, Ww300, bw300):
    h, x_pad, mf = pl.pallas_call(
        _mask_kernel,
        out_shape=[
            jax.ShapeDtypeStruct((NP, D), jnp.float32),
            jax.ShapeDtypeStruct((NP, D), jnp.float32),
            jax.ShapeDtypeStruct((NP, D), jnp.float32),
        ],
    )(x, node_mask)
    src = edge_index[0]
    dst = edge_index[1]
    pad = EPAD - E
    src_p = jnp.concatenate([src, jnp.zeros((pad,), jnp.int32)])
    dst_p = jnp.concatenate([dst, jnp.full((pad,), N, jnp.int32)]).reshape(NW, NCHUNK, CH)
    zrows = jnp.zeros((RPT, D), jnp.float32)
    batch_pad = jnp.concatenate([batch, jnp.full((NP - N,), PAD_GID, jnp.int32)])
    brow = batch_pad.reshape(NBLK, CH)
    bcol = batch_pad.reshape(NP, 1)
    r = lambda v: v.reshape(1, -1)
    for l in range(L):
        parts = _edge_agg_call(h, src_p, dst_p, zrows)
        h = _layer_call[l](
            h, parts, brow, bcol,
            W1[l], r(b1[l]), W2[l], r(b2[l]),
            r(gn1_alpha[l]), r(gn1_gamma[l]), r(gn1_beta[l]),
            r(gn2_alpha[l if l < L - 1 else 0]),
            r(gn2_gamma[l if l < L - 1 else 0]),
            r(gn2_beta[l if l < L - 1 else 0]),
        )
    class_out, loss = _final_call(
        h, x_pad, mf, brow, bcol,
        Wdec, r(bdec), Wfc1, r(bfc1), Wfc2, r(bfc2),
        Wecfp, r(becfp), Wtor, r(btor), Ww300, r(bw300),
        ecfp, torsion, word300_mean,
    )
    return class_out, loss[0, 0]


# speedup vs baseline: 2.4714x; 1.0503x over previous
import functools

import jax
import jax.numpy as jnp
from jax import lax
from jax.experimental import pallas as pl
from jax.experimental.pallas import tpu as pltpu
from jax.experimental.pallas import tpu_sc as plsc

N = 10000
E = 320000
D = 128
H = 128
G = 512
L = 4

NC = 2
NS = 16
NW = NC * NS
NP = 10112
RPT = NP // NS
EPT = 10240
EPAD = NW * EPT
CH = 128
NCHUNK = EPT // CH


HCH = NCHUNK // 2
HALF = HCH * CH
NGJ = HCH // 2


def _edge_agg_body(h_hbm, srcp_hbm, dstp_hbm, zrows_hbm, out_hbm,
                   sidx, didx, rowsA, rowsB, agg_sh,
                   sgA, sgB, ssA, ssB):
    c = lax.axis_index("c")
    s = lax.axis_index("s")
    wid = s * NC + c
    pltpu.sync_copy(zrows_hbm, agg_sh.at[pl.ds(s * RPT, RPT)])
    plsc.subcore_barrier()

    def gidx(ch):
        return sidx.at[pl.ds(ch * CH, CH)]

    for half in range(2):
        pltpu.sync_copy(srcp_hbm.at[pl.ds(wid * EPT + half * HALF, HALF)],
                        sidx)
        pltpu.sync_copy(dstp_hbm.at[wid, pl.ds(half * HCH, HCH)], didx)
        pltpu.async_copy(h_hbm.at[gidx(0)], rowsA, sgA)
        pltpu.async_copy(h_hbm.at[gidx(1)], rowsB, sgB)

        def step(j, carry):
            cA = 2 * j
            cB = 2 * j + 1
            pltpu.make_async_copy(h_hbm.at[gidx(cA)], rowsA, sgA).wait()
            pltpu.async_copy(rowsA, agg_sh.at[didx.at[cA]], ssA, add=True)
            pltpu.make_async_copy(h_hbm.at[gidx(cB)], rowsB, sgB).wait()
            pltpu.async_copy(rowsB, agg_sh.at[didx.at[cB]], ssB, add=True)

            @pl.when(j < NGJ - 1)
            def _():
                pltpu.make_async_copy(rowsA, agg_sh.at[didx.at[cA]],
                                      ssA).wait()
                pltpu.async_copy(h_hbm.at[gidx(cA + 2)], rowsA, sgA)
                pltpu.make_async_copy(rowsB, agg_sh.at[didx.at[cB]],
                                      ssB).wait()
                pltpu.async_copy(h_hbm.at[gidx(cB + 2)], rowsB, sgB)

            @pl.when(j == NGJ - 1)
            def _():
                pltpu.make_async_copy(rowsA, agg_sh.at[didx.at[cA]],
                                      ssA).wait()
                pltpu.make_async_copy(rowsB, agg_sh.at[didx.at[cB]],
                                      ssB).wait()

            return carry

        lax.fori_loop(0, NGJ, step, 0)

    plsc.subcore_barrier()
    pltpu.sync_copy(agg_sh.at[pl.ds(s * RPT, RPT)],
                    out_hbm.at[c, pl.ds(s * RPT, RPT)])


_edge_agg_call = functools.partial(
    pl.kernel,
    out_type=jax.ShapeDtypeStruct((NC, NP, D), jnp.float32),
    mesh=plsc.VectorSubcoreMesh(core_axis_name="c", subcore_axis_name="s"),
    scratch_types=[
        pltpu.VMEM((HALF,), jnp.int32),
        pltpu.VMEM((HCH, CH), jnp.int32),
        pltpu.VMEM((CH, D), jnp.float32),
        pltpu.VMEM((CH, D), jnp.float32),
        pltpu.VMEM_SHARED((NP, D), jnp.float32),
        pltpu.SemaphoreType.DMA,
        pltpu.SemaphoreType.DMA,
        pltpu.SemaphoreType.DMA,
        pltpu.SemaphoreType.DMA,
    ],
)(_edge_agg_body)


NBLK = NP // CH
PAD_GID = 520


def _mask_kernel(x_ref, m_ref, h0_ref, xp_ref, mf_ref):
    mf = m_ref[...].astype(jnp.float32)
    h0_ref[pl.ds(0, N), :] = jnp.where(m_ref[...], 0.0, x_ref[...])
    h0_ref[pl.ds(N, NP - N), :] = jnp.zeros((NP - N, D), jnp.float32)
    xp_ref[pl.ds(0, N), :] = x_ref[...]
    xp_ref[pl.ds(N, NP - N), :] = jnp.zeros((NP - N, D), jnp.float32)
    mf_ref[pl.ds(0, N), :] = mf
    mf_ref[pl.ds(N, NP - N), :] = jnp.zeros((NP - N, D), jnp.float32)


_GIDS = None


def _dot(a, b):
    return jax.lax.dot_general(a, b, (((1,), (0,)), ((), ())),
                               precision=jax.lax.Precision.HIGHEST,
                               preferred_element_type=jnp.float32)


def _gn_inplace(z_ref, brow_ref, bcol_ref, s12_ref, st_ref, cnt_ref,
                alpha, gamma, beta):
    s12_ref[...] = jnp.zeros_like(s12_ref)
    cnt_ref[...] = jnp.zeros_like(cnt_ref)
    gl_col = jax.lax.broadcasted_iota(jnp.int32, (G, 1), 0)
    gl_row = jax.lax.broadcasted_iota(jnp.int32, (1, G), 1)

    def red(b, carry):
        rb = b * CH
        zb = z_ref[pl.ds(rb, CH), :]
        bb = brow_ref[pl.ds(b, 1), :]
        A = (gl_col == bb).astype(jnp.float32)
        zcat = jnp.concatenate([zb, zb * zb], axis=1)
        s12_ref[...] = s12_ref[...] + _dot(A, zcat)
        cnt_ref[...] = cnt_ref[...] + jnp.sum(A, axis=1, keepdims=True)
        return carry

    jax.lax.fori_loop(0, NBLK, red, 0)

    c = jnp.maximum(cnt_ref[...], 1.0)
    S = s12_ref[...]
    mean = S[:, :H] / c
    ez2 = S[:, H:] / c
    var = ez2 + (alpha * alpha - 2.0 * alpha) * mean * mean
    sg = gamma / jnp.sqrt(var + 1e-5)
    tg = beta - alpha * mean * sg
    st_ref[...] = jnp.concatenate([sg, tg], axis=1)

    def exp_(b, carry):
        rb = b * CH
        bc = bcol_ref[pl.ds(rb, CH), :]
        B = (bc == gl_row).astype(jnp.float32)
        E = _dot(B, st_ref[...])
        zb = z_ref[pl.ds(rb, CH), :]
        zn = zb * E[:, :H] + E[:, H:]
        z_ref[pl.ds(rb, CH), :] = jnp.maximum(zn, 0.01 * zn)
        return carry

    jax.lax.fori_loop(0, NBLK, exp_, 0)


def _layer_body_mk(last):
    def body(h_ref, parts_ref, brow_ref, bcol_ref,
             W1_ref, b1_ref, W2_ref, b2_ref,
             a1_ref, g1_ref, be1_ref, a2_ref, g2_ref, be2_ref,
             out_ref, z_ref, s12_ref, st_ref, cnt_ref):
        z_ref[...] = _dot(h_ref[...] + parts_ref[0] + parts_ref[1],
                          W1_ref[...]) + b1_ref[...]
        _gn_inplace(z_ref, brow_ref, bcol_ref, s12_ref, st_ref, cnt_ref,
                    a1_ref[...], g1_ref[...], be1_ref[...])
        out_ref[...] = _dot(z_ref[...], W2_ref[...]) + b2_ref[...]
        if not last:
            _gn_inplace(out_ref, brow_ref, bcol_ref, s12_ref, st_ref, cnt_ref,
                        a2_ref[...], g2_ref[...], be2_ref[...])
    return body


_layer_call = [
    pl.pallas_call(
        _layer_body_mk(l == L - 1),
        out_shape=jax.ShapeDtypeStruct((NP, H), jnp.float32),
        scratch_shapes=[
            pltpu.VMEM((NP, H), jnp.float32),
            pltpu.VMEM((G, 2 * H), jnp.float32),
            pltpu.VMEM((G, 2 * H), jnp.float32),
            pltpu.VMEM((G, 1), jnp.float32),
        ],
    )
    for l in range(L)
]


def _final_body(h_ref, xp_ref, mf_ref, brow_ref, bcol_ref,
                Wdec_ref, bdec_ref, Wfc1_ref, bfc1_ref, Wfc2_ref, bfc2_ref,
                Wecfp_ref, becfp_ref, Wtor_ref, btor_ref, Ww300_ref, bw300_ref,
                ecfp_ref, tor_ref, w300_ref,
                cls_ref, loss_ref, sp_ref, mx_ref, cnt_ref):
    sp_ref[...] = jnp.zeros_like(sp_ref)
    cnt_ref[...] = jnp.zeros_like(cnt_ref)
    mx_ref[...] = jnp.full_like(mx_ref, -3.0e38)
    gl_col = jax.lax.broadcasted_iota(jnp.int32, (G, 1), 0)

    def red(b, carry):
        rb = b * CH
        hb = h_ref[pl.ds(rb, CH), :]
        bb = brow_ref[pl.ds(b, 1), :]
        bc = bcol_ref[pl.ds(rb, CH), :]
        A = (gl_col == bb).astype(jnp.float32)
        sp_ref[...] = sp_ref[...] + _dot(A, hb)
        cnt_ref[...] = cnt_ref[...] + jnp.sum(A, axis=1, keepdims=True)
        m = hb
        for k in (1, 2, 4, 8, 16, 32, 64):
            shifted = jnp.concatenate([m[:k], m[:-k]], axis=0)
            prev = jnp.concatenate(
                [jnp.full((k, 1), -7, jnp.int32), bc[:-k]], axis=0)
            same = bc == prev
            m = jnp.where(same, jnp.maximum(m, shifted), m)
        nxt = jnp.concatenate([bb[:, 1:], jnp.full((1, 1), -7, jnp.int32)],
                              axis=1)
        lastm = (bb != nxt).astype(jnp.float32)
        Alast = A * lastm
        P = _dot(Alast, m)
        pres = jnp.sum(Alast, axis=1, keepdims=True) > 0.0
        mx_ref[...] = jnp.where(pres, jnp.maximum(mx_ref[...], P), mx_ref[...])
        return carry

    jax.lax.fori_loop(0, NBLK, red, 0)

    counts = cnt_ref[...]
    c = jnp.maximum(counts, 1.0)
    add_p = sp_ref[...]
    mean_p = add_p / c
    max_p = jnp.where(counts > 0.0, mx_ref[...], 0.0)

    pooled = jnp.concatenate([mean_p, add_p, max_p], axis=1)
    pooled = _dot(pooled, Wfc1_ref[...]) + bfc1_ref[...]
    pooled = jnp.maximum(pooled, 0.01 * pooled)
    cls_ref[...] = (_dot(pooled, Wfc2_ref[...]) + bfc2_ref[...]) / 2.0

    recon = _dot(h_ref[...], Wdec_ref[...]) + bdec_ref[...]
    dlt = (recon - xp_ref[...]) * mf_ref[...]
    pre_loss = jnp.sum(dlt * dlt) / jnp.maximum(jnp.sum(mf_ref[...]), 1.0)

    e1 = _dot(pooled, Wecfp_ref[...]) + becfp_ref[...] - ecfp_ref[...]
    ecfp_loss = jnp.sum(e1 * e1) / (G * 2048)
    e2 = _dot(pooled, Wtor_ref[...]) + btor_ref[...] - tor_ref[...]
    tor_loss = jnp.sum(e2 * e2) / (G * 2048)
    e3 = _dot(pooled, Ww300_ref[...]) + bw300_ref[...] - w300_ref[...]
    w300_loss = jnp.sum(e3 * e3) / (G * 300)
    loss_ref[...] = jnp.full((1, 1), 0.0) + pre_loss + ecfp_loss \
        + tor_loss + w300_loss


_final_call = pl.pallas_call(
    _final_body,
    out_shape=[
        jax.ShapeDtypeStruct((G, 3), jnp.float32),
        jax.ShapeDtypeStruct((1, 1), jnp.float32),
    ],
    scratch_shapes=[
        pltpu.VMEM((G, H), jnp.float32),
        pltpu.VMEM((G, H), jnp.float32),
        pltpu.VMEM((G, 1), jnp.float32),
    ],
)


def kernel(x, edge_index, edge_attr, batch, ecfp, word300_mean, torsion, node_mask, edge_mask, W1, b1, W2, b2, gn1_alpha, gn1_gamma, gn1_beta, gn2_alpha, gn2_gamma, gn2_beta, Wdec, bdec, Wfc1, bfc1, Wfc2, bfc2, Wecfp, becfp, Wtor, btor, Ww300, bw300):
    h, x_pad, mf = pl.pallas_call(
        _mask_kernel,
        out_shape=[
            jax.ShapeDtypeStruct((NP, D), jnp.float32),
            jax.ShapeDtypeStruct((NP, D), jnp.float32),
            jax.ShapeDtypeStruct((NP, D), jnp.float32),
        ],
    )(x, node_mask)
    src = edge_index[0]
    dst = edge_index[1]
    pad = EPAD - E
    src_p = jnp.concatenate([src, jnp.zeros((pad,), jnp.int32)])
    dst_p = jnp.concatenate([dst, jnp.full((pad,), N, jnp.int32)]).reshape(NW, NCHUNK, CH)
    zrows = jnp.zeros((RPT, D), jnp.float32)
    batch_pad = jnp.concatenate([batch, jnp.full((NP - N,), PAD_GID, jnp.int32)])
    brow = batch_pad.reshape(NBLK, CH)
    bcol = batch_pad.reshape(NP, 1)
    r = lambda v: v.reshape(1, -1)
    for l in range(L):
        parts = _edge_agg_call(h, src_p, dst_p, zrows)
        h = _layer_call[l](
            h, parts, brow, bcol,
            W1[l], r(b1[l]), W2[l], r(b2[l]),
            r(gn1_alpha[l]), r(gn1_gamma[l]), r(gn1_beta[l]),
            r(gn2_alpha[l if l < L - 1 else 0]),
            r(gn2_gamma[l if l < L - 1 else 0]),
            r(gn2_beta[l if l < L - 1 else 0]),
        )
    class_out, loss = _final_call(
        h, x_pad, mf, brow, bcol,
        Wdec, r(bdec), Wfc1, r(bfc1), Wfc2, r(bfc2),
        Wecfp, r(becfp), Wtor, r(btor), Ww300, r(bw300),
        ecfp, torsion, word300_mean,
    )
    return class_out, loss[0, 0]
